# baseline (speedup 1.0000x reference)
import functools

import numpy as np
import jax
import jax.numpy as jnp
from jax import lax
from jax.experimental import pallas as pl
from jax.experimental.pallas import tpu as pltpu
from jax.experimental.pallas import tpu_sc as plsc

N = 64
NSCALE = 4
_B, _D = 4, 256
_NC = 2


_EA = 33
_EB = N + 1 - _EA


def _zero_range(plane, e_lo, e_hi, s_loc, zero):
    @pl.loop(e_lo, e_hi)
    def _(e):
        for q in range(8):
            plane[e, s_loc, pl.ds(q * 16, 16)] = zero


def _sc_hidden_body(xt, out, xbuf, pa0, pa1, pb, sa0, sa1, sb):
    wid = lax.axis_index("s") * _NC + lax.axis_index("c")
    b = wid // 8
    rem = wid % 8
    r = rem // 2
    dh = rem % 2
    beta = 1 << r

    pltpu.sync_copy(xt.at[b, r, dh], xbuf)

    zero = jnp.zeros((16,), jnp.float32)
    pas = (pa0, pa1)
    sas = (sa0, sa1)

    @pl.loop(0, _EA)
    def _(e):
        for s_loc in range(8):
            for j in range(8):
                pa0[e, s_loc, pl.ds(j * 16, 16)] = zero
                pa1[e, s_loc, pl.ds(j * 16, 16)] = zero

    @pl.loop(0, _EB)
    def _(e):
        for s_loc in range(8):
            for j in range(8):
                pb[e, s_loc, pl.ds(j * 16, 16)] = zero

    def band(i, pa, sem_a, reclaim_a, reclaim_b):
        d_slice = out.at[b, r, pl.ds(0, _EA), pl.ds(i * 8, 8),
                         pl.ds(dh * 128, 128)]
        d_sliceb = out.at[b, r, pl.ds(_EA, _EB), pl.ds(i * 8, 8),
                          pl.ds(dh * 128, 128)]

        @pl.when(reclaim_a)
        def _():
            pltpu.make_async_copy(pa, d_slice, sem_a).wait()
            for s_loc in range(8):
                lo = jnp.maximum((i - 2) * 8 + s_loc + 1, 0)
                hi = jnp.minimum(i * 8 + s_loc, _EA - 1) + 1
                _zero_range(pa, lo, hi, s_loc, zero)

        @pl.when(reclaim_b)
        def _():
            pltpu.make_async_copy(pb, d_sliceb, sb).wait()
            for s_loc in range(8):
                lo = jnp.maximum((i - 1) * 8 + s_loc + 1 - _EA, 0)
                hi = jnp.maximum(jnp.minimum(i * 8 + s_loc - _EA,
                                             _EB - 1) + 1, 0)
                _zero_range(pb, lo, hi, s_loc, zero)

        for s_loc in range(8):
            s = i * 8 + s_loc

            @pl.when(lax.rem(s, beta) == 0)
            def _():
                lmax = (N - s) // beta
                l1 = jnp.maximum((_EA - 1 - s) // beta, 0)
                m0 = tuple(xbuf[s, pl.ds(j * 16, 16)] for j in range(8))

                def step(l, m, plane, e_off):
                    row = s + l - 1
                    e = s + l * beta - e_off
                    new = tuple(
                        jnp.maximum(m[j], xbuf[row, pl.ds(j * 16, 16)])
                        for j in range(8))
                    for j in range(8):
                        plane[e, s_loc, pl.ds(j * 16, 16)] = new[j]
                    return new

                m1 = pl.loop(1, l1 + 1, init_carry=m0)(
                    lambda l, m: step(l, m, pa, 0))
                if m1 is None:
                    m1 = m0
                pl.loop(l1 + 1, lmax + 1, init_carry=m1)(
                    lambda l, m: step(l, m, pb, _EA))

        pltpu.async_copy(pa, d_slice, sem_a)
        pltpu.async_copy(pb, d_sliceb, sb)

    @pl.loop(0, N // 8, step=2)
    def _(i):
        band(i, pa0, sa0, i > 1, i > 0)
        band(i + 1, pa1, sa1, i > 0, i >= 0)

    tail = out.at[b, r, pl.ds(0, _EA), pl.ds(0, 8), pl.ds(dh * 128, 128)]
    tailb = out.at[b, r, pl.ds(_EA, _EB), pl.ds(0, 8), pl.ds(dh * 128, 128)]
    pltpu.make_async_copy(pa0, tail, sa0).wait()
    pltpu.make_async_copy(pa1, tail, sa1).wait()
    pltpu.make_async_copy(pb, tailb, sb).wait()


_E_BLK = 5


def _tc_mask_body(msk):
    ec = pl.program_id(0)
    s_i = lax.broadcasted_iota(jnp.int32, (N, _D), 0)
    for r in range(NSCALE):
        beta = 1 << r
        for de in range(_E_BLK):
            e = ec * _E_BLK + de
            m = (((s_i & (beta - 1)) == 0) & (e > s_i)
                 & (((e - s_i) & (beta - 1)) == 0)).astype(jnp.float32)
            for b in range(_B):
                msk[b, r, de] = m


@jax.jit
def _run(xt):
    hid_p = pl.kernel(
        _sc_hidden_body,
        out_type=jax.ShapeDtypeStruct((_B, NSCALE, N + 1, N, _D),
                                      jnp.float32),
        mesh=plsc.VectorSubcoreMesh(core_axis_name="c", subcore_axis_name="s"),
        compiler_params=pltpu.CompilerParams(needs_layout_passes=False,
                                             use_tc_tiling_on_sc=True),
        scratch_types=[
            pltpu.VMEM((N, 128), jnp.float32),
            pltpu.VMEM((_EA, 8, 128), jnp.float32),
            pltpu.VMEM((_EA, 8, 128), jnp.float32),
            pltpu.VMEM((_EB, 8, 128), jnp.float32),
            pltpu.SemaphoreType.DMA,
            pltpu.SemaphoreType.DMA,
            pltpu.SemaphoreType.DMA,
        ],
    )(xt)
    msk_p = pl.pallas_call(
        _tc_mask_body,
        grid=((N + 1) // _E_BLK,),
        out_shape=jax.ShapeDtypeStruct((_B, NSCALE, N + 1, N, _D),
                                       jnp.float32),
        out_specs=pl.BlockSpec((_B, NSCALE, _E_BLK, N, _D),
                               lambda ec: (0, 0, ec, 0, 0)),
    )()
    perm = (0, 4, 1, 3, 2)
    return jnp.transpose(hid_p, perm), jnp.transpose(msk_p, perm)


def kernel(x):
    xt = x[:, :NSCALE].reshape(_B, NSCALE, 2, 128, N).transpose(0, 1, 2, 4, 3)
    return _run(xt)


# speedup vs baseline: 1.0877x; 1.0877x over previous
import functools

import numpy as np
import jax
import jax.numpy as jnp
from jax import lax
from jax.experimental import pallas as pl
from jax.experimental.pallas import tpu as pltpu
from jax.experimental.pallas import tpu_sc as plsc

N = 64
NSCALE = 4
_B, _D = 4, 256
_NC = 2


def _sc_hidden_body(xt, out, xbuf, plane, sem_x):
    wid = lax.axis_index("s") * _NC + lax.axis_index("c")
    b = wid // 8
    rem = wid % 8
    r = rem // 2
    dh = rem % 2
    beta = 1 << r

    xcp = pltpu.async_copy(xt.at[b, r, dh], xbuf, sem_x)

    zero = jnp.zeros((16,), jnp.float32)

    @pl.loop(0, N + 1)
    def _(e):
        for s_loc in range(8):
            for j in range(8):
                plane[e, s_loc, pl.ds(j * 16, 16)] = zero

    xcp.wait()

    @pl.loop(0, N // 8)
    def _(i):
        @pl.when(i > 0)
        def _():
            for s_loc in range(8):
                for j in range(1, 9):
                    e = (i - 1) * 8 + s_loc + j
                    for q in range(8):
                        plane[e, s_loc, pl.ds(q * 16, 16)] = zero

        for s_loc in range(8):
            s = i * 8 + s_loc

            @pl.when(lax.rem(s, beta) == 0)
            def _():
                lmax = (N - s) // beta
                m0 = tuple(xbuf[s, pl.ds(j * 16, 16)] for j in range(8))

                def inner(l, m):
                    row = s + l - 1
                    e = s + l * beta
                    new = tuple(
                        jnp.maximum(m[j], xbuf[row, pl.ds(j * 16, 16)])
                        for j in range(8))
                    for j in range(8):
                        plane[e, s_loc, pl.ds(j * 16, 16)] = new[j]
                    return new

                pl.loop(1, lmax + 1, init_carry=m0)(inner)

        pltpu.sync_copy(
            plane,
            out.at[b, r, slice(None), pl.ds(i * 8, 8), pl.ds(dh * 128, 128)])


_E_BLK = 5


def _tc_mask_body(msk):
    ec = pl.program_id(0)
    s_i = lax.broadcasted_iota(jnp.int32, (N, _D), 0)
    for r in range(NSCALE):
        beta = 1 << r
        for de in range(_E_BLK):
            e = ec * _E_BLK + de
            m = (((s_i & (beta - 1)) == 0) & (e > s_i)
                 & (((e - s_i) & (beta - 1)) == 0)).astype(jnp.float32)
            for b in range(_B):
                msk[b, r, de] = m


@jax.jit
def _run(xt):
    hid_p = pl.kernel(
        _sc_hidden_body,
        out_type=jax.ShapeDtypeStruct((_B, NSCALE, N + 1, N, _D),
                                      jnp.float32),
        mesh=plsc.VectorSubcoreMesh(core_axis_name="c", subcore_axis_name="s"),
        compiler_params=pltpu.CompilerParams(needs_layout_passes=False,
                                             use_tc_tiling_on_sc=True),
        scratch_types=[
            pltpu.VMEM((N, 128), jnp.float32),
            pltpu.VMEM((N + 1, 8, 128), jnp.float32),
            pltpu.SemaphoreType.DMA,
        ],
    )(xt)
    msk_p = pl.pallas_call(
        _tc_mask_body,
        grid=((N + 1) // _E_BLK,),
        out_shape=jax.ShapeDtypeStruct((_B, NSCALE, N + 1, N, _D),
                                       jnp.float32),
        out_specs=pl.BlockSpec((_B, NSCALE, _E_BLK, N, _D),
                               lambda ec: (0, 0, ec, 0, 0)),
    )()
    perm = (0, 4, 1, 3, 2)
    return jnp.transpose(hid_p, perm), jnp.transpose(msk_p, perm)


def kernel(x):
    xt = x[:, :NSCALE].reshape(_B, NSCALE, 2, 128, N).transpose(0, 1, 2, 4, 3)
    return _run(xt)


# speedup vs baseline: 1.0905x; 1.0025x over previous
import jax
import jax.numpy as jnp
from jax import lax
from jax.experimental import pallas as pl
from jax.experimental.pallas import tpu as pltpu
from jax.experimental.pallas import tpu_sc as plsc

N = 64
NSCALE = 4
_B, _D = 4, 256
_NC = 2


def _sc_hidden_body(xt, out, xbuf, plane, sem_x):
    wid = lax.axis_index("s") * _NC + lax.axis_index("c")
    b = wid // 8
    rem = wid % 8
    r = rem // 2
    dh = rem % 2
    beta = 1 << r

    xcp = pltpu.async_copy(xt.at[b, r, dh], xbuf, sem_x)

    zero = jnp.zeros((16,), jnp.float32)

    @pl.loop(0, N + 1)
    def _(e):
        for s_loc in range(8):
            for j in range(8):
                plane[e, s_loc, pl.ds(j * 16, 16)] = zero

    xcp.wait()

    @pl.loop(0, N // 8)
    def _(i):
        @pl.when(i > 0)
        def _():
            for s_loc in range(8):
                for j in range(1, 9):
                    e = (i - 1) * 8 + s_loc + j
                    for q in range(8):
                        plane[e, s_loc, pl.ds(q * 16, 16)] = zero

        for s_loc in range(8):
            s = i * 8 + s_loc

            @pl.when(lax.rem(s, beta) == 0)
            def _():
                lmax = (N - s) // beta
                m0 = tuple(xbuf[s, pl.ds(j * 16, 16)] for j in range(8))

                def inner(l, m):
                    row = s + l - 1
                    e = s + l * beta
                    new = tuple(
                        jnp.maximum(m[j], xbuf[row, pl.ds(j * 16, 16)])
                        for j in range(8))
                    for j in range(8):
                        plane[e, s_loc, pl.ds(j * 16, 16)] = new[j]
                    return new

                pl.loop(1, lmax + 1, init_carry=m0)(inner)

        pltpu.sync_copy(
            plane,
            out.at[b, r, slice(None), pl.ds(i * 8, 8), pl.ds(dh * 128, 128)])


_E_BLK = 5


def _tc_mask_body(msk):
    ec = pl.program_id(0)
    s_i = lax.broadcasted_iota(jnp.int32, (N, _D), 0)
    for r in range(NSCALE):
        beta = 1 << r
        for de in range(_E_BLK):
            e = ec * _E_BLK + de
            m = (((s_i & (beta - 1)) == 0) & (e > s_i)
                 & (((e - s_i) & (beta - 1)) == 0)).astype(jnp.float32)
            for b in range(_B):
                msk[b, r, de] = m


@jax.jit
def _run(xt):
    hid_p = pl.kernel(
        _sc_hidden_body,
        out_type=jax.ShapeDtypeStruct((_B, NSCALE, N + 1, N, _D),
                                      jnp.float32),
        mesh=plsc.VectorSubcoreMesh(core_axis_name="c", subcore_axis_name="s"),
        compiler_params=pltpu.CompilerParams(needs_layout_passes=False,
                                             use_tc_tiling_on_sc=True),
        scratch_types=[
            pltpu.VMEM((N, 128), jnp.float32),
            pltpu.VMEM((N + 1, 8, 128), jnp.float32),
            pltpu.SemaphoreType.DMA,
        ],
    )(xt)
    msk_p = pl.pallas_call(
        _tc_mask_body,
        grid=((N + 1) // _E_BLK,),
        out_shape=jax.ShapeDtypeStruct((_B, NSCALE, N + 1, N, _D),
                                       jnp.float32),
        out_specs=pl.BlockSpec((_B, NSCALE, _E_BLK, N, _D),
                               lambda ec: (0, 0, ec, 0, 0)),
    )()
    perm = (0, 4, 1, 3, 2)
    return jnp.transpose(hid_p, perm), jnp.transpose(msk_p, perm)


def kernel(x):
    xt = x[:, :NSCALE].reshape(_B, NSCALE, 2, 128, N).transpose(0, 1, 2, 4, 3)
    return _run(xt)
